# Initial kernel scaffold; baseline (speedup 1.0000x reference)
import jax
import jax.numpy as jnp
from jax.experimental import pallas as pl


def kernel(x, edge_index, W_enc, b_enc, W_hid, b_hid, W_dec, b_dec):
    raise NotImplementedError("write your pallas kernel here")


# speedup vs baseline: 17.0858x; 17.0858x over previous
import functools

import jax
import jax.numpy as jnp
from jax import lax
from jax.experimental import pallas as pl
from jax.experimental.pallas import tpu as pltpu
from jax.experimental.pallas import tpu_sc as plsc

N_NODES = 10000
NP = 10240
N_EDGES = 320000
H = 128
NC = 2
NS = 16
NW = NC * NS
EPW = N_EDGES // NW
B = 125
NB = EPW // B
NROWS = N_EDGES // B
UNROLL = 4
RPT = NP // NS

_mesh = plsc.VectorSubcoreMesh(core_axis_name="c", subcore_axis_name="s")


def _deg_body(dst_hbm, ones_hbm, zeros_hbm, out_hbm,
              dst_v, ones_v, acc_sh, s0, s1, s2, s3):
    c = lax.axis_index("c")
    s = lax.axis_index("s")
    w = c * NS + s
    sems = (s0, s1, s2, s3)
    pltpu.sync_copy(zeros_hbm, acc_sh.at[pl.ds(s * RPT, RPT)])
    pltpu.sync_copy(ones_hbm, ones_v)
    pltpu.sync_copy(dst_hbm.at[pl.ds(w * NB, NB)], dst_v)
    plsc.subcore_barrier()

    def step(i, carry):
        descs = []
        for b in range(UNROLL):
            j = i * UNROLL + b
            descs.append(pltpu.async_copy(
                ones_v, acc_sh.at[dst_v.at[j]], sems[b], add=True))
        for d in descs:
            d.wait()
        return carry

    lax.fori_loop(0, NB // UNROLL, step, 0)
    plsc.subcore_barrier()
    pltpu.sync_copy(acc_sh.at[pl.ds(s * RPT, RPT)],
                    out_hbm.at[c].at[pl.ds(s * RPT, RPT)])


_sc_params = pltpu.CompilerParams(use_tc_tiling_on_sc=False)

_deg_call = pl.kernel(
    _deg_body,
    out_type=jax.ShapeDtypeStruct((NC, NP, 16), jnp.float32),
    mesh=_mesh,
    compiler_params=_sc_params,
    scratch_types=[
        pltpu.VMEM((NB, B), jnp.int32),
        pltpu.VMEM((B, 16), jnp.float32),
        pltpu.VMEM_SHARED((NP, 16), jnp.float32),
        pltpu.SemaphoreType.DMA,
        pltpu.SemaphoreType.DMA,
        pltpu.SemaphoreType.DMA,
        pltpu.SemaphoreType.DMA,
    ],
)


HH = H // 2
NBA = NROWS // NS


def _agg_body(v_hbm, srcA_hbm, srcB_hbm, dst_hbm, zeros_hbm, out_hbm,
              src_v, dst_v, rb0, rb1, rb2, rb3,
              g0, g1, g2, g3, t0, t1, t2, t3, acc_sh):
    c = lax.axis_index("c")
    s = lax.axis_index("s")
    rbufs = (rb0, rb1, rb2, rb3)
    gsems = (g0, g1, g2, g3)
    ssems = (t0, t1, t2, t3)
    pltpu.sync_copy(zeros_hbm, acc_sh.at[pl.ds(s * RPT, RPT)])

    @pl.when(c == 0)
    def _():
        pltpu.sync_copy(srcA_hbm.at[pl.ds(s * NBA, NBA)], src_v)

    @pl.when(c == 1)
    def _():
        pltpu.sync_copy(srcB_hbm.at[pl.ds(s * NBA, NBA)], src_v)

    pltpu.sync_copy(dst_hbm.at[pl.ds(s * NBA, NBA)], dst_v)
    plsc.subcore_barrier()

    def step(i, carry):
        gd = []
        for b in range(UNROLL):
            j = i * UNROLL + b
            gd.append(pltpu.async_copy(
                v_hbm.at[src_v.at[j]], rbufs[b], gsems[b]))
        sd = []
        for b in range(UNROLL):
            j = i * UNROLL + b
            gd[b].wait()
            sd.append(pltpu.async_copy(
                rbufs[b], acc_sh.at[dst_v.at[j]], ssems[b], add=True))
        for d in sd:
            d.wait()
        return carry

    lax.fori_loop(0, NBA // UNROLL, step, 0)
    plsc.subcore_barrier()
    pltpu.sync_copy(acc_sh.at[pl.ds(s * RPT, RPT)],
                    out_hbm.at[c].at[pl.ds(s * RPT, RPT)])


_agg_call = pl.kernel(
    _agg_body,
    out_type=jax.ShapeDtypeStruct((NC, NP, HH), jnp.float32),
    mesh=_mesh,
    compiler_params=_sc_params,
    scratch_types=[
        pltpu.VMEM((NBA, B), jnp.int32),
        pltpu.VMEM((NBA, B), jnp.int32),
        pltpu.VMEM((B, HH), jnp.float32),
        pltpu.VMEM((B, HH), jnp.float32),
        pltpu.VMEM((B, HH), jnp.float32),
        pltpu.VMEM((B, HH), jnp.float32),
        pltpu.SemaphoreType.DMA,
        pltpu.SemaphoreType.DMA,
        pltpu.SemaphoreType.DMA,
        pltpu.SemaphoreType.DMA,
        pltpu.SemaphoreType.DMA,
        pltpu.SemaphoreType.DMA,
        pltpu.SemaphoreType.DMA,
        pltpu.SemaphoreType.DMA,
        pltpu.VMEM_SHARED((NP, HH), jnp.float32),
    ],
)


_R = 1024
_GRID = NP // _R

_row_spec = pl.BlockSpec((_R, H), lambda i: (i, 0))
_gA_spec = pl.BlockSpec((1, _R, HH), lambda i: (0, i, 0))
_gB_spec = pl.BlockSpec((1, _R, HH), lambda i: (1, i, 0))
_dA_spec = pl.BlockSpec((1, _R, 16), lambda i: (0, i, 0))
_dB_spec = pl.BlockSpec((1, _R, 16), lambda i: (1, i, 0))
_dis_spec = pl.BlockSpec((_R, 1), lambda i: (i, 0))
_b_spec = pl.BlockSpec((1, H), lambda i: (0, 0))
_W_spec = pl.BlockSpec((H, H), lambda i: (0, 0))


def _prep_body(d0_ref, d1_ref, x_ref, W_ref, dis_ref, v_ref):
    deg = d0_ref[0, :, 0:1] + d1_ref[0, :, 0:1] + 1.0
    dis = lax.rsqrt(deg)
    dis_ref[...] = dis
    v_ref[...] = dis * jnp.dot(x_ref[...], W_ref[...],
                               preferred_element_type=jnp.float32)


_prep_call = pl.pallas_call(
    _prep_body,
    grid=(_GRID,),
    in_specs=[_dA_spec, _dB_spec, _row_spec, _W_spec],
    out_specs=[_dis_spec, _row_spec],
    out_shape=[
        jax.ShapeDtypeStruct((NP, 1), jnp.float32),
        jax.ShapeDtypeStruct((NP, H), jnp.float32),
    ],
)


def _layer_body(ga_ref, gb_ref, v_ref, dis_ref, b_ref, W_ref, out_ref):
    dis = dis_ref[...]
    g = jnp.concatenate([ga_ref[0], gb_ref[0]], axis=1)
    pre = dis * (g + v_ref[...]) + b_ref[...]
    h = jnp.maximum(pre, 0.0)
    out_ref[...] = dis * jnp.dot(h, W_ref[...],
                                 preferred_element_type=jnp.float32)


_layer_call = pl.pallas_call(
    _layer_body,
    grid=(_GRID,),
    in_specs=[_gA_spec, _gB_spec, _row_spec, _dis_spec, _b_spec, _W_spec],
    out_specs=_row_spec,
    out_shape=jax.ShapeDtypeStruct((NP, H), jnp.float32),
)


def _final_body(ga_ref, gb_ref, v_ref, dis_ref, b_ref, out_ref):
    g = jnp.concatenate([ga_ref[0], gb_ref[0]], axis=1)
    out_ref[...] = dis_ref[...] * (g + v_ref[...]) + b_ref[...]


_final_call = pl.pallas_call(
    _final_body,
    grid=(_GRID,),
    in_specs=[_gA_spec, _gB_spec, _row_spec, _dis_spec, _b_spec],
    out_specs=_row_spec,
    out_shape=jax.ShapeDtypeStruct((NP, H), jnp.float32),
)


def kernel(x, edge_index, W_enc, b_enc, W_hid, b_hid, W_dec, b_dec):
    src = edge_index[0]
    dst2 = edge_index[1].reshape(NROWS, B)
    srcA = (src * 2).reshape(NROWS, B)
    srcB = (src * 2 + 1).reshape(NROWS, B)

    xp = jnp.pad(x, ((0, NP - x.shape[0]), (0, H - x.shape[1])))
    Wep = jnp.pad(W_enc, ((0, H - W_enc.shape[0]), (0, 0)))
    Wdp = jnp.pad(W_dec, ((0, 0), (0, H - W_dec.shape[1])))
    bdp = jnp.pad(b_dec, (0, H - b_dec.shape[0]))

    zeros_h = jnp.zeros((RPT, HH), jnp.float32)
    zeros16 = jnp.zeros((RPT, 16), jnp.float32)
    ones16 = jnp.ones((B, 16), jnp.float32)

    degs = _deg_call(dst2, ones16, zeros16)
    dis, v = _prep_call(degs, degs, xp, Wep)

    b_enc2 = b_enc.reshape(1, H)
    b_hid2 = b_hid.reshape(1, H)
    bdp2 = bdp.reshape(1, H)

    weights = [W_hid, W_hid, W_hid, W_hid, Wdp]
    biases = [b_enc2, b_hid2, b_hid2, b_hid2, b_hid2]
    for l in range(5):
        g = _agg_call(v.reshape(2 * NP, HH), srcA, srcB, dst2, zeros_h)
        v = _layer_call(g, g, v, dis, biases[l], weights[l])
    g = _agg_call(v.reshape(2 * NP, HH), srcA, srcB, dst2, zeros_h)
    out = _final_call(g, g, v, dis, bdp2)
    return out[:N_NODES, :3]


# speedup vs baseline: 20.3688x; 1.1921x over previous
import functools

import jax
import jax.numpy as jnp
from jax import lax
from jax.experimental import pallas as pl
from jax.experimental.pallas import tpu as pltpu
from jax.experimental.pallas import tpu_sc as plsc

N_NODES = 10000
NP = 10240
N_EDGES = 320000
H = 128
NC = 2
NS = 16
NW = NC * NS
EPW = N_EDGES // NW
B = 125
NB = EPW // B
NROWS = N_EDGES // B
UNROLL = 4
RPT = NP // NS

_mesh = plsc.VectorSubcoreMesh(core_axis_name="c", subcore_axis_name="s")


def _deg_body(dst_hbm, ones_hbm, zeros_hbm, out_hbm,
              dst_v, ones_v, acc_sh, s0, s1, s2, s3):
    c = lax.axis_index("c")
    s = lax.axis_index("s")
    w = c * NS + s
    sems = (s0, s1, s2, s3)
    pltpu.sync_copy(zeros_hbm, acc_sh.at[pl.ds(s * RPT, RPT)])
    pltpu.sync_copy(ones_hbm, ones_v)
    pltpu.sync_copy(dst_hbm.at[pl.ds(w * NB, NB)], dst_v)
    plsc.subcore_barrier()

    def step(i, carry):
        descs = []
        for b in range(UNROLL):
            j = i * UNROLL + b
            descs.append(pltpu.async_copy(
                ones_v, acc_sh.at[dst_v.at[j]], sems[b], add=True))
        for d in descs:
            d.wait()
        return carry

    lax.fori_loop(0, NB // UNROLL, step, 0)
    plsc.subcore_barrier()
    pltpu.sync_copy(acc_sh.at[pl.ds(s * RPT, RPT)],
                    out_hbm.at[c].at[pl.ds(s * RPT, RPT)])


_sc_params = pltpu.CompilerParams(use_tc_tiling_on_sc=False)

_deg_call = pl.kernel(
    _deg_body,
    out_type=jax.ShapeDtypeStruct((NC, NP, 16), jnp.float32),
    mesh=_mesh,
    compiler_params=_sc_params,
    scratch_types=[
        pltpu.VMEM((NB, B), jnp.int32),
        pltpu.VMEM((B, 16), jnp.float32),
        pltpu.VMEM_SHARED((NP, 16), jnp.float32),
        pltpu.SemaphoreType.DMA,
        pltpu.SemaphoreType.DMA,
        pltpu.SemaphoreType.DMA,
        pltpu.SemaphoreType.DMA,
    ],
)


HH = H // 2
NBA = NROWS // NS


NBUF = 4


def _agg_body(v_hbm, srcA_hbm, srcB_hbm, dst_hbm, zeros_hbm, out_hbm,
              src_v, dst_v,
              rb0, rb1, rb2, rb3,
              g0, g1, g2, g3,
              t0, t1, t2, t3, acc_sh):
    rbufs = (rb0, rb1, rb2, rb3)
    gsems = (g0, g1, g2, g3)
    ssems = (t0, t1, t2, t3)
    c = lax.axis_index("c")
    s = lax.axis_index("s")
    pltpu.sync_copy(zeros_hbm, acc_sh.at[pl.ds(s * RPT, RPT)])

    @pl.when(c == 0)
    def _():
        pltpu.sync_copy(srcA_hbm.at[pl.ds(s * NBA, NBA)], src_v)

    @pl.when(c == 1)
    def _():
        pltpu.sync_copy(srcB_hbm.at[pl.ds(s * NBA, NBA)], src_v)

    pltpu.sync_copy(dst_hbm.at[pl.ds(s * NBA, NBA)], dst_v)
    plsc.subcore_barrier()

    for b in range(NBUF):
        pltpu.async_copy(v_hbm.at[src_v.at[b]], rbufs[b], gsems[b])

    def step(i, carry):
        base = i * NBUF
        for b in range(NBUF):
            j = base + b
            pltpu.make_async_copy(
                v_hbm.at[src_v.at[j]], rbufs[b], gsems[b]).wait()
            pltpu.async_copy(
                rbufs[b], acc_sh.at[dst_v.at[j]], ssems[b], add=True)
        for b in range(NBUF):
            j = base + b
            pltpu.make_async_copy(
                rbufs[b], acc_sh.at[dst_v.at[j]], ssems[b]).wait()
            jn = j + NBUF

            @pl.when(jn < NBA)
            def _():
                pltpu.async_copy(v_hbm.at[src_v.at[jn]], rbufs[b], gsems[b])

        return carry

    lax.fori_loop(0, NBA // NBUF, step, 0)
    plsc.subcore_barrier()
    pltpu.sync_copy(acc_sh.at[pl.ds(s * RPT, RPT)],
                    out_hbm.at[c].at[pl.ds(s * RPT, RPT)])


_agg_call = pl.kernel(
    _agg_body,
    out_type=jax.ShapeDtypeStruct((NC, NP, HH), jnp.float32),
    mesh=_mesh,
    compiler_params=_sc_params,
    scratch_types=[
        pltpu.VMEM((NBA, B), jnp.int32),
        pltpu.VMEM((NBA, B), jnp.int32),
        *[pltpu.VMEM((B, HH), jnp.float32) for _ in range(NBUF)],
        *[pltpu.SemaphoreType.DMA for _ in range(2 * NBUF)],
        pltpu.VMEM_SHARED((NP, HH), jnp.float32),
    ],
)


_R = 1024
_GRID = NP // _R

_row_spec = pl.BlockSpec((_R, H), lambda i: (i, 0))
_gA_spec = pl.BlockSpec((1, _R, HH), lambda i: (0, i, 0))
_gB_spec = pl.BlockSpec((1, _R, HH), lambda i: (1, i, 0))
_dA_spec = pl.BlockSpec((1, _R, 16), lambda i: (0, i, 0))
_dB_spec = pl.BlockSpec((1, _R, 16), lambda i: (1, i, 0))
_dis_spec = pl.BlockSpec((_R, 1), lambda i: (i, 0))
_b_spec = pl.BlockSpec((1, H), lambda i: (0, 0))
_W_spec = pl.BlockSpec((H, H), lambda i: (0, 0))


def _prep_body(d0_ref, d1_ref, x_ref, W_ref, dis_ref, v_ref):
    deg = d0_ref[0, :, 0:1] + d1_ref[0, :, 0:1] + 1.0
    dis = lax.rsqrt(deg)
    dis_ref[...] = dis
    v_ref[...] = dis * jnp.dot(x_ref[...], W_ref[...],
                               preferred_element_type=jnp.float32)


_prep_call = pl.pallas_call(
    _prep_body,
    grid=(_GRID,),
    in_specs=[_dA_spec, _dB_spec, _row_spec, _W_spec],
    out_specs=[_dis_spec, _row_spec],
    out_shape=[
        jax.ShapeDtypeStruct((NP, 1), jnp.float32),
        jax.ShapeDtypeStruct((NP, H), jnp.float32),
    ],
)


def _layer_body(ga_ref, gb_ref, v_ref, dis_ref, b_ref, W_ref, out_ref):
    dis = dis_ref[...]
    g = jnp.concatenate([ga_ref[0], gb_ref[0]], axis=1)
    pre = dis * (g + v_ref[...]) + b_ref[...]
    h = jnp.maximum(pre, 0.0)
    out_ref[...] = dis * jnp.dot(h, W_ref[...],
                                 preferred_element_type=jnp.float32)


_layer_call = pl.pallas_call(
    _layer_body,
    grid=(_GRID,),
    in_specs=[_gA_spec, _gB_spec, _row_spec, _dis_spec, _b_spec, _W_spec],
    out_specs=_row_spec,
    out_shape=jax.ShapeDtypeStruct((NP, H), jnp.float32),
)


def _final_body(ga_ref, gb_ref, v_ref, dis_ref, b_ref, out_ref):
    g = jnp.concatenate([ga_ref[0], gb_ref[0]], axis=1)
    out_ref[...] = dis_ref[...] * (g + v_ref[...]) + b_ref[...]


_final_call = pl.pallas_call(
    _final_body,
    grid=(_GRID,),
    in_specs=[_gA_spec, _gB_spec, _row_spec, _dis_spec, _b_spec],
    out_specs=_row_spec,
    out_shape=jax.ShapeDtypeStruct((NP, H), jnp.float32),
)


def kernel(x, edge_index, W_enc, b_enc, W_hid, b_hid, W_dec, b_dec):
    src = edge_index[0]
    dst2 = edge_index[1].reshape(NROWS, B)
    srcA = (src * 2).reshape(NROWS, B)
    srcB = (src * 2 + 1).reshape(NROWS, B)

    xp = jnp.pad(x, ((0, NP - x.shape[0]), (0, H - x.shape[1])))
    Wep = jnp.pad(W_enc, ((0, H - W_enc.shape[0]), (0, 0)))
    Wdp = jnp.pad(W_dec, ((0, 0), (0, H - W_dec.shape[1])))
    bdp = jnp.pad(b_dec, (0, H - b_dec.shape[0]))

    zeros_h = jnp.zeros((RPT, HH), jnp.float32)
    zeros16 = jnp.zeros((RPT, 16), jnp.float32)
    ones16 = jnp.ones((B, 16), jnp.float32)

    degs = _deg_call(dst2, ones16, zeros16)
    dis, v = _prep_call(degs, degs, xp, Wep)

    b_enc2 = b_enc.reshape(1, H)
    b_hid2 = b_hid.reshape(1, H)
    bdp2 = bdp.reshape(1, H)

    weights = [W_hid, W_hid, W_hid, W_hid, Wdp]
    biases = [b_enc2, b_hid2, b_hid2, b_hid2, b_hid2]
    for l in range(5):
        g = _agg_call(v.reshape(2 * NP, HH), srcA, srcB, dst2, zeros_h)
        v = _layer_call(g, g, v, dis, biases[l], weights[l])
    g = _agg_call(v.reshape(2 * NP, HH), srcA, srcB, dst2, zeros_h)
    out = _final_call(g, g, v, dis, bdp2)
    return out[:N_NODES, :3]
